# baseline (speedup 1.0000x reference)
import functools

import jax
import jax.numpy as jnp
from jax import lax
from jax.experimental import pallas as pl
from jax.experimental.pallas import tpu as pltpu
from jax.experimental.pallas import tpu_sc as plsc

N_GENE = 50000
N_GO = 10000
E = 300000
H = 128

NC = 2
NS = 16
NW = NC * NS

GROUP = 128
GPT = 80
E_PAD = NW * GPT * GROUP
NGROUPS = E_PAD // GROUP

GO_TILE = 640
GO_PAD = GO_TILE * NS
NR = 5
RNG = 10240
GENE_TILE = 3136
GENE_CPAD = GENE_TILE * NS
TAB_PAD = 10240

_MESH = plsc.VectorSubcoreMesh(
    core_axis_name="c", subcore_axis_name="s", num_cores=NC, num_subcores=NS)


def _pad_edges(ei, n_dst, spread):
  src = ei[0].astype(jnp.int32)
  dst = ei[1].astype(jnp.int32)
  pad = E_PAD - E
  src_p = jnp.concatenate([src, jnp.zeros((pad,), jnp.int32)])
  dst_p = jnp.concatenate(
      [dst, n_dst + (jnp.arange(pad, dtype=jnp.int32) % spread)])
  return src_p.reshape(NGROUPS, GROUP), dst_p.reshape(NGROUPS, GROUP)


def _pad_rows(x):
  return jnp.pad(x, ((0, TAB_PAD - x.shape[0]), (0, 0)))


def _fill_2d(ref, nrows, ncols, value):
  z = jnp.full((16,), value, jnp.float32)
  kchunks = ncols // 16
  def b(i, _):
    ref[i // kchunks, pl.ds((i % kchunks) * 16, 16)] = z
    return 0
  lax.fori_loop(0, nrows * kchunks, b, 0)


def _zero_fill_2d(ref, nrows, ncols):
  _fill_2d(ref, nrows, ncols, 0.0)


def _zero_fill_1d(ref, n):
  z = jnp.zeros((16,), jnp.float32)
  def b(i, _):
    ref[pl.ds(i * 16, 16)] = z
    return 0
  lax.fori_loop(0, n // 16, b, 0)


def _spmem_fill(buf, nbuf, acc, base, nrows):
  off = 0
  while off < nrows:
    n = min(nbuf, nrows - off)
    pltpu.sync_copy(buf.at[pl.ds(0, n)], acc.at[pl.ds(base + off, n)])
    off += n


_CH_GO = GO_PAD // H
_CH_GENE = (NR * RNG) // H
_RED = 10


def _cnt_body(dst_go, dst_gene, ident, out_go, out_gene,
              hgo, hgene, didx, dr_v, dc_v, rows, sem_g, sem_s):
  c = lax.axis_index("c")
  s = lax.axis_index("s")
  wid = c * NS + s
  g0 = wid * GPT
  _zero_fill_2d(rows, 64, H)
  @pl.when(s < _RED)
  def _():
    pltpu.sync_copy(rows.at[pl.ds(0, _CH_GO // _RED)],
                    hgo.at[pl.ds(s * (_CH_GO // _RED), _CH_GO // _RED)])
    pltpu.sync_copy(rows.at[pl.ds(0, _CH_GENE // _RED)],
                    hgene.at[pl.ds(s * (_CH_GENE // _RED), _CH_GENE // _RED)])
  plsc.subcore_barrier()

  def do_type(dst2d, hist):
    def chunk(k, _):
      pltpu.sync_copy(dst2d.at[pl.ds(g0 + k * 16, 16)], didx)
      def rd(i, _):
        r = i // 8
        col = (i % 8) * 16
        d = didx[r, pl.ds(col, 16)]
        dr_v[r, pl.ds(col, 16)] = lax.shift_right_logical(d, 7)
        dc_v[r, pl.ds(col, 16)] = lax.bitwise_and(d, 127)
        return 0
      lax.fori_loop(0, 16 * 8, rd, 0)
      def blk(i, _):
        gds = []
        for j in range(2):
          g = i * 2 + j
          gds.append(pltpu.async_copy(
              ident.at[dc_v.at[g]], rows.at[pl.ds(j * GROUP, GROUP)], sem_g))
        for d in gds:
          d.wait()
        sds = []
        for j in range(2):
          g = i * 2 + j
          sds.append(pltpu.async_copy(
              rows.at[pl.ds(j * GROUP, GROUP)], hist.at[dr_v.at[g]], sem_s,
              add=True))
        for d in sds:
          d.wait()
        return 0
      lax.fori_loop(0, 8, blk, 0)
      return 0
    lax.fori_loop(0, GPT // 16, chunk, 0)

  do_type(dst_go, hgo)
  do_type(dst_gene, hgene)
  plsc.subcore_barrier()
  @pl.when(s < _RED)
  def _():
    ngo = _CH_GO // _RED
    pltpu.sync_copy(hgo.at[pl.ds(s * ngo, ngo)], rows.at[pl.ds(0, ngo)])
    pltpu.sync_copy(rows.at[pl.ds(0, ngo)], out_go.at[c, pl.ds(s * ngo, ngo)])
    ngene = _CH_GENE // _RED
    pltpu.sync_copy(hgene.at[pl.ds(s * ngene, ngene)],
                    rows.at[pl.ds(0, ngene)])
    pltpu.sync_copy(rows.at[pl.ds(0, ngene)],
                    out_gene.at[c, pl.ds(s * ngene, ngene)])


def _cnt_call(dst_go2d, dst_gene2d):
  ident = jnp.eye(H, dtype=jnp.float32)
  fn = pl.kernel(
      _cnt_body,
      out_type=(jax.ShapeDtypeStruct((NC, _CH_GO, H), jnp.float32),
                jax.ShapeDtypeStruct((NC, _CH_GENE, H), jnp.float32)),
      mesh=_MESH,
      scratch_types=[
          pltpu.VMEM_SHARED((_CH_GO, H), jnp.float32),
          pltpu.VMEM_SHARED((_CH_GENE, H), jnp.float32),
          pltpu.VMEM((16, GROUP), jnp.int32),
          pltpu.VMEM((16, GROUP), jnp.int32),
          pltpu.VMEM((16, GROUP), jnp.int32),
          pltpu.VMEM((2 * GROUP, H), jnp.float32),
          pltpu.SemaphoreType.DMA,
          pltpu.SemaphoreType.DMA,
      ],
  )
  return fn(dst_go2d, dst_gene2d, ident)


def _go_agg_body(src2d, dst2d, table, out,
                 acc, sidx, didx, rows, sem_g, sem_s):
  c = lax.axis_index("c")
  s = lax.axis_index("s")
  wid = c * NS + s
  base = s * GO_TILE
  _zero_fill_2d(rows, 2 * GROUP, H)
  _spmem_fill(rows, 2 * GROUP, acc, base, GO_TILE)
  g0 = wid * GPT
  plsc.subcore_barrier()

  def chunk(k, _):
    goff = g0 + k * 16
    pltpu.sync_copy(src2d.at[pl.ds(goff, 16)], sidx)
    pltpu.sync_copy(dst2d.at[pl.ds(goff, 16)], didx)
    def blk(i, _):
      gds = []
      for j in range(2):
        g = i * 2 + j
        gds.append(pltpu.async_copy(
            table.at[sidx.at[g]], rows.at[pl.ds(j * GROUP, GROUP)], sem_g))
      for d in gds:
        d.wait()
      sds = []
      for j in range(2):
        g = i * 2 + j
        sds.append(pltpu.async_copy(
            rows.at[pl.ds(j * GROUP, GROUP)], acc.at[didx.at[g]], sem_s,
            add=True))
      for d in sds:
        d.wait()
      return 0
    lax.fori_loop(0, 8, blk, 0)
    return 0
  lax.fori_loop(0, GPT // 16, chunk, 0)
  plsc.subcore_barrier()
  off = 0
  while off < GO_TILE:
    n = min(2 * GROUP, GO_TILE - off)
    pltpu.sync_copy(acc.at[pl.ds(base + off, n)], rows.at[pl.ds(0, n)])
    pltpu.sync_copy(rows.at[pl.ds(0, n)],
                    out.at[c, pl.ds(base + off, n)])
    off += n


def _go_agg_call(src2d, dst2d, table):
  fn = pl.kernel(
      _go_agg_body,
      out_type=jax.ShapeDtypeStruct((NC, GO_PAD, H), jnp.float32),
      mesh=_MESH,
      scratch_types=[
          pltpu.VMEM_SHARED((GO_PAD, H), jnp.float32),
          pltpu.VMEM((16, GROUP), jnp.int32),
          pltpu.VMEM((16, GROUP), jnp.int32),
          pltpu.VMEM((2 * GROUP, H), jnp.float32),
          pltpu.SemaphoreType.DMA,
          pltpu.SemaphoreType.DMA,
      ],
  )
  return fn(src2d, dst2d, table)


def _gene_agg_body(src2d, dst2d, table, out,
                   acc, sidx, didx, didx2, rows, sem_g, sem_s):
  c = lax.axis_index("c")
  s = lax.axis_index("s")
  wid = c * NS + s
  base = s * GO_TILE
  g0 = wid * GPT
  for rk in range(NR):
    rbase = rk * RNG
    _zero_fill_2d(rows, 2 * GROUP, H)
    _spmem_fill(rows, 2 * GROUP, acc, base, GO_TILE)
    plsc.subcore_barrier()

    def chunk(k, _):
      goff = g0 + k * 16
      pltpu.sync_copy(src2d.at[pl.ds(goff, 16)], sidx)
      pltpu.sync_copy(dst2d.at[pl.ds(goff, 16)], didx)
      def rd(i, _):
        r = i // 8
        col = (i % 8) * 16
        d = didx[r, pl.ds(col, 16)]
        inr = (d >= rbase) & (d < rbase + RNG)
        dump = RNG + lax.bitwise_and(d, 1023)
        didx2[r, pl.ds(col, 16)] = jnp.where(inr, d - rbase, dump)
        sv = sidx[r, pl.ds(col, 16)]
        sidx[r, pl.ds(col, 16)] = jnp.where(inr, sv, 0)
        return 0
      lax.fori_loop(0, 16 * 8, rd, 0)
      def blk(i, _):
        gds = []
        for j in range(2):
          g = i * 2 + j
          gds.append(pltpu.async_copy(
              table.at[sidx.at[g]], rows.at[pl.ds(j * GROUP, GROUP)], sem_g))
        for d in gds:
          d.wait()
        sds = []
        for j in range(2):
          g = i * 2 + j
          sds.append(pltpu.async_copy(
              rows.at[pl.ds(j * GROUP, GROUP)], acc.at[didx2.at[g]], sem_s,
              add=True))
        for d in sds:
          d.wait()
        return 0
      lax.fori_loop(0, 8, blk, 0)
      return 0
    lax.fori_loop(0, GPT // 16, chunk, 0)
    plsc.subcore_barrier()
    off = 0
    while off < GO_TILE:
      n = min(2 * GROUP, GO_TILE - off)
      pltpu.sync_copy(acc.at[pl.ds(base + off, n)], rows.at[pl.ds(0, n)])
      pltpu.sync_copy(rows.at[pl.ds(0, n)],
                      out.at[c, rk, pl.ds(base + off, n)])
      off += n


def _gene_agg_call(src2d, dst2d, table):
  fn = pl.kernel(
      _gene_agg_body,
      out_type=jax.ShapeDtypeStruct((NC, NR, RNG, H), jnp.float32),
      mesh=_MESH,
      scratch_types=[
          pltpu.VMEM_SHARED((RNG + 1024, H), jnp.float32),
          pltpu.VMEM((16, GROUP), jnp.int32),
          pltpu.VMEM((16, GROUP), jnp.int32),
          pltpu.VMEM((16, GROUP), jnp.int32),
          pltpu.VMEM((2 * GROUP, H), jnp.float32),
          pltpu.SemaphoreType.DMA,
          pltpu.SemaphoreType.DMA,
      ],
  )
  return fn(src2d, dst2d, table)


_BR = 1000


def _dot(a, b):
  return jax.lax.dot_general(
      a, b, (((1,), (0,)), ((), ())),
      precision=jax.lax.Precision.HIGHEST,
      preferred_element_type=jnp.float32)


def _upd_kernel(relu, aggp, cnt, x, wl, wr, b, out):
  a = aggp[0] + aggp[1]
  cn = cnt[0] + cnt[1]
  inv = 1.0 / jnp.maximum(cn, 1.0)
  h = _dot(a * inv, wl[...]) + b[...] + _dot(x[...], wr[...])
  out[...] = jnp.maximum(h, 0.0) if relu else h


def _upd_call(aggp, cnt3, x, wl, wr, b, relu, n):
  grid = n // _BR
  return pl.pallas_call(
      functools.partial(_upd_kernel, relu),
      grid=(grid,),
      in_specs=[
          pl.BlockSpec((NC, _BR, H), lambda i: (0, i, 0)),
          pl.BlockSpec((NC, _BR, 1), lambda i: (0, i, 0)),
          pl.BlockSpec((_BR, H), lambda i: (i, 0)),
          pl.BlockSpec((H, H), lambda i: (0, 0)),
          pl.BlockSpec((H, H), lambda i: (0, 0)),
          pl.BlockSpec((1, H), lambda i: (0, 0)),
      ],
      out_specs=pl.BlockSpec((_BR, H), lambda i: (i, 0)),
      out_shape=jax.ShapeDtypeStruct((n, H), jnp.float32),
  )(aggp, cnt3, x, wl, wr, b)


def kernel(x_gene, x_go, edge_index_gene_to_go, edge_index_go_to_gene,
           W1l_g2go, b1_g2go, W1r_g2go, W1l_go2g, b1_go2g, W1r_go2g,
           W2l_g2go, b2_g2go, W2r_g2go, W2l_go2g, b2_go2g, W2r_go2g):
  src_g2go, dst_g2go = _pad_edges(edge_index_gene_to_go, N_GO, GO_PAD - N_GO)
  src_go2g, dst_go2g = _pad_edges(edge_index_go_to_gene, N_GENE, 1024)

  cnt_go, cnt_gene = _cnt_call(dst_g2go, dst_go2g)
  cnt_go3 = cnt_go.reshape(NC, GO_PAD, 1)
  cnt_gene3 = cnt_gene.reshape(NC, NR * RNG, 1)

  agg_go1 = _go_agg_call(src_g2go, dst_g2go, x_gene)
  agg_gene1 = _gene_agg_call(src_go2g, dst_go2g, _pad_rows(x_go))
  agg_gene1 = agg_gene1.reshape(NC, NR * RNG, H)

  h_go = _upd_call(agg_go1, cnt_go3, x_go, W1l_g2go, W1r_g2go,
                   b1_g2go[None, :], True, N_GO)
  h_gene = _upd_call(agg_gene1, cnt_gene3, x_gene, W1l_go2g, W1r_go2g,
                     b1_go2g[None, :], True, N_GENE)

  agg_go2 = _go_agg_call(src_g2go, dst_g2go, h_gene)
  agg_gene2 = _gene_agg_call(src_go2g, dst_go2g, _pad_rows(h_go))
  agg_gene2 = agg_gene2.reshape(NC, NR * RNG, H)

  z_go = _upd_call(agg_go2, cnt_go3, h_go, W2l_g2go, W2r_g2go,
                   b2_g2go[None, :], False, N_GO)
  z_gene = _upd_call(agg_gene2, cnt_gene3, h_gene, W2l_go2g, W2r_go2g,
                     b2_go2g[None, :], False, N_GENE)
  return (z_gene, z_go)


# speedup vs baseline: 8.0109x; 8.0109x over previous
import functools

import jax
import jax.numpy as jnp
from jax import lax
from jax.experimental import pallas as pl
from jax.experimental.pallas import tpu as pltpu
from jax.experimental.pallas import tpu_sc as plsc

N_GENE = 50000
N_GO = 10000
E = 300000
H = 128

NC = 2
NS = 16
NW = NC * NS

GROUP = 128
GPT = 80
E_PAD = NW * GPT * GROUP
NGROUPS = E_PAD // GROUP

GO_TILE = 640
GO_PAD = GO_TILE * NS
NR = 4
RNG = 12800
GTILE = RNG // NS
GENE_TILE = 3136
GENE_CPAD = GENE_TILE * NS
TAB_PAD = 10240

_MESH = plsc.VectorSubcoreMesh(
    core_axis_name="c", subcore_axis_name="s", num_cores=NC, num_subcores=NS)


def _pad_edges(ei, n_dst, spread):
  src = ei[0].astype(jnp.int32)
  dst = ei[1].astype(jnp.int32)
  pad = E_PAD - E
  src_p = jnp.concatenate([src, jnp.zeros((pad,), jnp.int32)])
  dst_p = jnp.concatenate(
      [dst, n_dst + (jnp.arange(pad, dtype=jnp.int32) % spread)])
  return src_p.reshape(NGROUPS, GROUP), dst_p.reshape(NGROUPS, GROUP)


def _pad_rows(x):
  return jnp.pad(x, ((0, TAB_PAD - x.shape[0]), (0, 0)))


def _fill_2d(ref, nrows, ncols, value):
  z = jnp.full((16,), value, jnp.float32)
  kchunks = ncols // 16
  def b(i, _):
    ref[i // kchunks, pl.ds((i % kchunks) * 16, 16)] = z
    return 0
  lax.fori_loop(0, nrows * kchunks, b, 0)


def _zero_fill_2d(ref, nrows, ncols):
  _fill_2d(ref, nrows, ncols, 0.0)


def _zero_fill_1d(ref, n):
  z = jnp.zeros((16,), jnp.float32)
  def b(i, _):
    ref[pl.ds(i * 16, 16)] = z
    return 0
  lax.fori_loop(0, n // 16, b, 0)


def _spmem_fill(buf, nbuf, acc, base, nrows):
  off = 0
  while off < nrows:
    n = min(nbuf, nrows - off)
    pltpu.sync_copy(buf.at[pl.ds(0, n)], acc.at[pl.ds(base + off, n)])
    off += n


_CH_GO = GO_PAD // H
_CH_GENE = (NR * RNG) // H
_RED = 10


def _cnt_body(dst_go, dst_gene, ident, out_go, out_gene,
              hgo, hgene, didx, dr_v, dc_v, rows, sem_g, sem_s):
  c = lax.axis_index("c")
  s = lax.axis_index("s")
  wid = c * NS + s
  g0 = wid * GPT
  _zero_fill_2d(rows, 64, H)
  @pl.when(s < _RED)
  def _():
    pltpu.sync_copy(rows.at[pl.ds(0, _CH_GO // _RED)],
                    hgo.at[pl.ds(s * (_CH_GO // _RED), _CH_GO // _RED)])
    pltpu.sync_copy(rows.at[pl.ds(0, _CH_GENE // _RED)],
                    hgene.at[pl.ds(s * (_CH_GENE // _RED), _CH_GENE // _RED)])
  plsc.subcore_barrier()

  def do_type(dst2d, hist):
    def chunk(k, _):
      pltpu.sync_copy(dst2d.at[pl.ds(g0 + k * 16, 16)], didx)
      def rd(i, _):
        r = i // 8
        col = (i % 8) * 16
        d = didx[r, pl.ds(col, 16)]
        dr_v[r, pl.ds(col, 16)] = lax.shift_right_logical(d, 7)
        dc_v[r, pl.ds(col, 16)] = lax.bitwise_and(d, 127)
        return 0
      lax.fori_loop(0, 16 * 8, rd, 0)
      def blk(i, _):
        gds = []
        for j in range(2):
          g = i * 2 + j
          gds.append(pltpu.async_copy(
              ident.at[dc_v.at[g]], rows.at[pl.ds(j * GROUP, GROUP)], sem_g))
        for d in gds:
          d.wait()
        sds = []
        for j in range(2):
          g = i * 2 + j
          sds.append(pltpu.async_copy(
              rows.at[pl.ds(j * GROUP, GROUP)], hist.at[dr_v.at[g]], sem_s,
              add=True))
        for d in sds:
          d.wait()
        return 0
      lax.fori_loop(0, 8, blk, 0)
      return 0
    lax.fori_loop(0, GPT // 16, chunk, 0)

  do_type(dst_go, hgo)
  do_type(dst_gene, hgene)
  plsc.subcore_barrier()
  @pl.when(s < _RED)
  def _():
    ngo = _CH_GO // _RED
    pltpu.sync_copy(hgo.at[pl.ds(s * ngo, ngo)], rows.at[pl.ds(0, ngo)])
    pltpu.sync_copy(rows.at[pl.ds(0, ngo)], out_go.at[c, pl.ds(s * ngo, ngo)])
    ngene = _CH_GENE // _RED
    pltpu.sync_copy(hgene.at[pl.ds(s * ngene, ngene)],
                    rows.at[pl.ds(0, ngene)])
    pltpu.sync_copy(rows.at[pl.ds(0, ngene)],
                    out_gene.at[c, pl.ds(s * ngene, ngene)])


def _cnt_call(dst_go2d, dst_gene2d):
  ident = jnp.eye(H, dtype=jnp.float32)
  fn = pl.kernel(
      _cnt_body,
      out_type=(jax.ShapeDtypeStruct((NC, _CH_GO, H), jnp.float32),
                jax.ShapeDtypeStruct((NC, _CH_GENE, H), jnp.float32)),
      mesh=_MESH,
      scratch_types=[
          pltpu.VMEM_SHARED((_CH_GO, H), jnp.float32),
          pltpu.VMEM_SHARED((_CH_GENE, H), jnp.float32),
          pltpu.VMEM((16, GROUP), jnp.int32),
          pltpu.VMEM((16, GROUP), jnp.int32),
          pltpu.VMEM((16, GROUP), jnp.int32),
          pltpu.VMEM((2 * GROUP, H), jnp.float32),
          pltpu.SemaphoreType.DMA,
          pltpu.SemaphoreType.DMA,
      ],
  )
  return fn(dst_go2d, dst_gene2d, ident)


def _go_agg_body(src2d, dst2d, table, out,
                 acc, sidx, didx, rows, sem_g, sem_s):
  c = lax.axis_index("c")
  s = lax.axis_index("s")
  wid = c * NS + s
  base = s * GO_TILE
  _zero_fill_2d(rows, 2 * GROUP, H)
  _spmem_fill(rows, 2 * GROUP, acc, base, GO_TILE)
  g0 = wid * GPT
  plsc.subcore_barrier()

  def chunk(k, _):
    goff = g0 + k * 16
    pltpu.sync_copy(src2d.at[pl.ds(goff, 16)], sidx)
    pltpu.sync_copy(dst2d.at[pl.ds(goff, 16)], didx)
    def blk(i, _):
      gds = []
      for j in range(2):
        g = i * 2 + j
        gds.append(pltpu.async_copy(
            table.at[sidx.at[g]], rows.at[pl.ds(j * GROUP, GROUP)], sem_g))
      for d in gds:
        d.wait()
      sds = []
      for j in range(2):
        g = i * 2 + j
        sds.append(pltpu.async_copy(
            rows.at[pl.ds(j * GROUP, GROUP)], acc.at[didx.at[g]], sem_s,
            add=True))
      for d in sds:
        d.wait()
      return 0
    lax.fori_loop(0, 8, blk, 0)
    return 0
  lax.fori_loop(0, GPT // 16, chunk, 0)
  plsc.subcore_barrier()
  off = 0
  while off < GO_TILE:
    n = min(2 * GROUP, GO_TILE - off)
    pltpu.sync_copy(acc.at[pl.ds(base + off, n)], rows.at[pl.ds(0, n)])
    pltpu.sync_copy(rows.at[pl.ds(0, n)],
                    out.at[c, pl.ds(base + off, n)])
    off += n


def _go_agg_call(src2d, dst2d, table):
  fn = pl.kernel(
      _go_agg_body,
      out_type=jax.ShapeDtypeStruct((NC, GO_PAD, H), jnp.float32),
      mesh=_MESH,
      scratch_types=[
          pltpu.VMEM_SHARED((GO_PAD, H), jnp.float32),
          pltpu.VMEM((16, GROUP), jnp.int32),
          pltpu.VMEM((16, GROUP), jnp.int32),
          pltpu.VMEM((2 * GROUP, H), jnp.float32),
          pltpu.SemaphoreType.DMA,
          pltpu.SemaphoreType.DMA,
      ],
  )
  return fn(src2d, dst2d, table)


def _gene_agg_body(src2d, dst2d, table, out,
                   acc, sidx, didx, rows, sem_g, sem_s):
  c = lax.axis_index("c")
  s = lax.axis_index("s")
  wid = c * NS + s
  base = s * GTILE
  g0 = wid * GPT
  for rk in range(NR):
    rbase = rk * RNG
    _zero_fill_2d(rows, GROUP, H)
    _spmem_fill(rows, GROUP, acc, base, GTILE)
    plsc.subcore_barrier()

    def chunk(k, _):
      goff = g0 + k * 16
      pltpu.sync_copy(src2d.at[pl.ds(goff, 16)], sidx)
      pltpu.sync_copy(dst2d.at[pl.ds(goff, 16)], didx)
      def rd(i, _):
        r = i // 8
        col = (i % 8) * 16
        d = didx[r, pl.ds(col, 16)]
        inr = (d >= rbase) & (d < rbase + RNG)
        dump = RNG + lax.bitwise_and(d, 511)
        didx[r, pl.ds(col, 16)] = jnp.where(inr, d - rbase, dump)
        return 0
      lax.fori_loop(0, 16 * 8, rd, 0)
      def blk(g, _):
        pltpu.async_copy(table.at[sidx.at[g]], rows, sem_g).wait()
        pltpu.async_copy(rows, acc.at[didx.at[g]], sem_s, add=True).wait()
        return 0
      lax.fori_loop(0, 16, blk, 0)
      return 0
    lax.fori_loop(0, GPT // 16, chunk, 0)
    plsc.subcore_barrier()
    off = 0
    while off < GTILE:
      n = min(GROUP, GTILE - off)
      pltpu.sync_copy(acc.at[pl.ds(base + off, n)], rows.at[pl.ds(0, n)])
      pltpu.sync_copy(rows.at[pl.ds(0, n)],
                      out.at[c, rk, pl.ds(base + off, n)])
      off += n


def _gene_agg_call(src2d, dst2d, table):
  fn = pl.kernel(
      _gene_agg_body,
      out_type=jax.ShapeDtypeStruct((NC, NR, RNG, H), jnp.float32),
      mesh=_MESH,
      scratch_types=[
          pltpu.VMEM_SHARED((RNG + 512, H), jnp.float32),
          pltpu.VMEM((16, GROUP), jnp.int32),
          pltpu.VMEM((16, GROUP), jnp.int32),
          pltpu.VMEM((GROUP, H), jnp.float32),
          pltpu.SemaphoreType.DMA,
          pltpu.SemaphoreType.DMA,
      ],
  )
  return fn(src2d, dst2d, table)


_BR = 1000


def _dot(a, b):
  return jax.lax.dot_general(
      a, b, (((1,), (0,)), ((), ())),
      precision=jax.lax.Precision.HIGHEST,
      preferred_element_type=jnp.float32)


def _upd_kernel(relu, aggp, cnt, x, wl, wr, b, out):
  a = aggp[0] + aggp[1]
  cn = cnt[0] + cnt[1]
  inv = 1.0 / jnp.maximum(cn, 1.0)
  h = _dot(a * inv, wl[...]) + b[...] + _dot(x[...], wr[...])
  out[...] = jnp.maximum(h, 0.0) if relu else h


def _upd_call(aggp, cnt3, x, wl, wr, b, relu, n):
  grid = n // _BR
  return pl.pallas_call(
      functools.partial(_upd_kernel, relu),
      grid=(grid,),
      in_specs=[
          pl.BlockSpec((NC, _BR, H), lambda i: (0, i, 0)),
          pl.BlockSpec((NC, _BR, 1), lambda i: (0, i, 0)),
          pl.BlockSpec((_BR, H), lambda i: (i, 0)),
          pl.BlockSpec((H, H), lambda i: (0, 0)),
          pl.BlockSpec((H, H), lambda i: (0, 0)),
          pl.BlockSpec((1, H), lambda i: (0, 0)),
      ],
      out_specs=pl.BlockSpec((_BR, H), lambda i: (i, 0)),
      out_shape=jax.ShapeDtypeStruct((n, H), jnp.float32),
  )(aggp, cnt3, x, wl, wr, b)


def kernel(x_gene, x_go, edge_index_gene_to_go, edge_index_go_to_gene,
           W1l_g2go, b1_g2go, W1r_g2go, W1l_go2g, b1_go2g, W1r_go2g,
           W2l_g2go, b2_g2go, W2r_g2go, W2l_go2g, b2_go2g, W2r_go2g):
  src_g2go, dst_g2go = _pad_edges(edge_index_gene_to_go, N_GO, GO_PAD - N_GO)
  src_go2g, dst_go2g = _pad_edges(edge_index_go_to_gene, N_GENE, 1024)

  cnt_go, cnt_gene = _cnt_call(dst_g2go, dst_go2g)
  cnt_go3 = cnt_go.reshape(NC, GO_PAD, 1)
  cnt_gene3 = cnt_gene.reshape(NC, NR * RNG, 1)

  agg_go1 = _go_agg_call(src_g2go, dst_g2go, x_gene)
  agg_gene1 = _gene_agg_call(src_go2g, dst_go2g, _pad_rows(x_go))
  agg_gene1 = agg_gene1.reshape(NC, NR * RNG, H)

  h_go = _upd_call(agg_go1, cnt_go3, x_go, W1l_g2go, W1r_g2go,
                   b1_g2go[None, :], True, N_GO)
  h_gene = _upd_call(agg_gene1, cnt_gene3, x_gene, W1l_go2g, W1r_go2g,
                     b1_go2g[None, :], True, N_GENE)

  agg_go2 = _go_agg_call(src_g2go, dst_g2go, h_gene)
  agg_gene2 = _gene_agg_call(src_go2g, dst_go2g, _pad_rows(h_go))
  agg_gene2 = agg_gene2.reshape(NC, NR * RNG, H)

  z_go = _upd_call(agg_go2, cnt_go3, h_go, W2l_g2go, W2r_g2go,
                   b2_g2go[None, :], False, N_GO)
  z_gene = _upd_call(agg_gene2, cnt_gene3, h_gene, W2l_go2g, W2r_go2g,
                     b2_go2g[None, :], False, N_GENE)
  return (z_gene, z_go)
